# baseline (speedup 1.0000x reference)
import functools
import math

import jax
import jax.numpy as jnp
from jax import lax
from jax.experimental import pallas as pl
from jax.experimental.pallas import tpu as pltpu
from jax.experimental.pallas import tpu_sc as plsc

NC = 2
NS = 16
CH = 128
RB = 400


def _sc_mesh():
    return plsc.VectorSubcoreMesh(core_axis_name="c", subcore_axis_name="s",
                                  num_cores=NC, num_subcores=NS)


@functools.lru_cache(maxsize=None)
def _make_deg_kernel(E_pad, AR):
    EPT = E_pad // (NC * NS)
    n_chunks = EPT // CH
    ZR = AR // NS

    @functools.partial(
        pl.kernel,
        out_type=[jax.ShapeDtypeStruct((AR, 16), jnp.float32),
                  jax.ShapeDtypeStruct((AR, 16), jnp.float32)],
        mesh=_sc_mesh(),
        scratch_types=[
            pltpu.VMEM((CH,), jnp.int32),
            pltpu.VMEM((CH, 16), jnp.float32),
            pltpu.VMEM((CH, 16), jnp.float32),
            pltpu.VMEM_SHARED((AR, 16), jnp.float32),
        ],
    )
    def deg_kernel(dst_hbm, out0_hbm, out1_hbm, dst_v, ones_v, zeros_v, acc):
        cid = lax.axis_index("c")
        sid = lax.axis_index("s")
        one = jnp.ones((16,), jnp.float32)
        zero = jnp.zeros((16,), jnp.float32)

        def fill(i, carry):
            ones_v[i, :] = one
            zeros_v[i, :] = zero
            return carry
        lax.fori_loop(0, CH, fill, 0)

        r0 = sid * ZR
        nfull = ZR // CH
        for t in range(nfull):
            pltpu.sync_copy(zeros_v, acc.at[pl.ds(r0 + t * CH, CH)])
        rem = ZR - nfull * CH
        if rem:
            pltpu.sync_copy(zeros_v.at[pl.ds(0, rem)],
                            acc.at[pl.ds(r0 + nfull * CH, rem)])
        plsc.subcore_barrier()

        ebase = (cid * NS + sid) * EPT

        def body(j, carry):
            pltpu.sync_copy(dst_hbm.at[pl.ds(ebase + j * CH, CH)], dst_v)
            pltpu.sync_copy(ones_v, acc.at[dst_v], add=True)
            return carry
        lax.fori_loop(0, n_chunks, body, 0)
        plsc.subcore_barrier()

        @pl.when(cid == 0)
        def _():
            pltpu.sync_copy(acc.at[pl.ds(r0, ZR)], out0_hbm.at[pl.ds(r0, ZR)])

        @pl.when(cid == 1)
        def _():
            pltpu.sync_copy(acc.at[pl.ds(r0, ZR)], out1_hbm.at[pl.ds(r0, ZR)])

    return deg_kernel


@functools.lru_cache(maxsize=None)
def _make_prop_kernel(E_pad, AR, D, K0, K1):
    TCH = E_pad // CH
    assert NS * (K0 + K1) == TCH and K0 >= K1
    ZR = AR // NS
    KM = K0

    @functools.partial(
        pl.kernel,
        out_type=[jax.ShapeDtypeStruct((AR, D), jnp.float32),
                  jax.ShapeDtypeStruct((AR, D), jnp.float32)],
        mesh=_sc_mesh(),
        scratch_types=[
            pltpu.VMEM((KM, CH), jnp.int32),
            pltpu.VMEM((KM, CH), jnp.int32),
            pltpu.VMEM((CH, D), jnp.float32),
            pltpu.VMEM((CH, D), jnp.float32),
            pltpu.VMEM_SHARED((AR, D), jnp.float32),
            pltpu.SemaphoreType.DMA,
            pltpu.SemaphoreType.DMA,
        ],
    )
    def prop_kernel(gs_hbm, src_hbm, dst_hbm, out0_hbm, out1_hbm,
                    src_all, dst_all, rows0, rows1, acc, gsem, ssem):
        cid = lax.axis_index("c")
        sid = lax.axis_index("s")
        zero = jnp.zeros((16,), jnp.float32)

        def run(k, cbase):
            pltpu.sync_copy(src_hbm.at[pl.ds(cbase, KM)], src_all)
            pltpu.sync_copy(dst_hbm.at[pl.ds(cbase, KM)], dst_all)

            def fill(i, carry):
                for q in range(D // 16):
                    rows0[i, pl.ds(q * 16, 16)] = zero
                return carry
            lax.fori_loop(0, CH, fill, 0)

            r0 = sid * ZR
            nfull = ZR // CH
            for t in range(nfull):
                pltpu.sync_copy(rows0, acc.at[pl.ds(r0 + t * CH, CH)])
            rem = ZR - nfull * CH
            if rem:
                pltpu.sync_copy(rows0.at[pl.ds(0, rem)],
                                acc.at[pl.ds(r0 + nfull * CH, rem)])
            plsc.subcore_barrier()

            gd = {}
            sd = {}
            gd[0] = pltpu.async_copy(gs_hbm.at[src_all.at[0]], rows0, gsem)
            for g in range(k):
                buf = rows0 if g % 2 == 0 else rows1
                nbuf = rows1 if g % 2 == 0 else rows0
                gd[g].wait()
                if g >= 1:
                    sd[g - 1].wait()
                if g + 1 < k:
                    gd[g + 1] = pltpu.async_copy(
                        gs_hbm.at[src_all.at[g + 1]], nbuf, gsem)
                sd[g] = pltpu.async_copy(
                    buf, acc.at[dst_all.at[g]], ssem, add=True)
            sd[k - 1].wait()
            plsc.subcore_barrier()

            r0 = sid * ZR

            @pl.when(cid == 0)
            def _():
                pltpu.sync_copy(acc.at[pl.ds(r0, ZR)], out0_hbm.at[pl.ds(r0, ZR)])

            @pl.when(cid == 1)
            def _():
                pltpu.sync_copy(acc.at[pl.ds(r0, ZR)], out1_hbm.at[pl.ds(r0, ZR)])

        if K0 == K1:
            run(K0, (cid * NS + sid) * K0)
        else:
            @pl.when(cid == 0)
            def _():
                run(K0, sid * K0)

            @pl.when(cid == 1)
            def _():
                run(K1, NS * K0 + sid * K1)

    return prop_kernel


def _stats_matmul_call(p0, p1, x, W1, N):
    IN = x.shape[1]
    H = W1.shape[1]
    grid = (N // RB,)

    def body(q0, q1, x_r, w1_r, m1_o, gs1_o, dis_o, d2_o):
        deg = q0[:, :1] + q1[:, :1] + 1.0
        dis = lax.rsqrt(deg)
        d2 = 1.0 / deg
        dis_b = jnp.broadcast_to(dis, (RB, H))
        m1 = jnp.dot(x_r[...], w1_r[...], preferred_element_type=jnp.float32)
        m1_o[...] = m1
        gs1_o[...] = dis_b * m1
        dis_o[...] = dis_b
        d2_o[...] = jnp.broadcast_to(d2, (RB, H))

    return pl.pallas_call(
        body,
        grid=grid,
        in_specs=[
            pl.BlockSpec((RB, 16), lambda i: (i, 0)),
            pl.BlockSpec((RB, 16), lambda i: (i, 0)),
            pl.BlockSpec((RB, IN), lambda i: (i, 0)),
            pl.BlockSpec((IN, H), lambda i: (0, 0)),
        ],
        out_specs=[
            pl.BlockSpec((RB, H), lambda i: (i, 0)),
            pl.BlockSpec((RB, H), lambda i: (i, 0)),
            pl.BlockSpec((RB, H), lambda i: (i, 0)),
            pl.BlockSpec((RB, H), lambda i: (i, 0)),
        ],
        out_shape=[
            jax.ShapeDtypeStruct((N, H), jnp.float32),
            jax.ShapeDtypeStruct((N, H), jnp.float32),
            jax.ShapeDtypeStruct((N, H), jnp.float32),
            jax.ShapeDtypeStruct((N, H), jnp.float32),
        ],
    )(p0, p1, x, W1)


def _combine_matmul_call(p0, p1, m_prev, dis, d2, b_prev, W_next, N, relu):
    H = m_prev.shape[1]
    HO = W_next.shape[1]
    grid = (N // RB,)

    def body(q0, q1, m_r, dis_r, d2_r, b_r, w_r, mn_o, gsn_o):
        comb = dis_r[...] * (q0[...] + q1[...]) + d2_r[...] * m_r[...] + b_r[...]
        if relu:
            comb = jnp.maximum(comb, 0.0)
        mn = jnp.dot(comb, w_r[...], preferred_element_type=jnp.float32)
        mn_o[...] = mn
        gsn_o[...] = dis_r[...] * mn

    return pl.pallas_call(
        body,
        grid=grid,
        in_specs=[
            pl.BlockSpec((RB, H), lambda i: (i, 0)),
            pl.BlockSpec((RB, H), lambda i: (i, 0)),
            pl.BlockSpec((RB, H), lambda i: (i, 0)),
            pl.BlockSpec((RB, H), lambda i: (i, 0)),
            pl.BlockSpec((RB, H), lambda i: (i, 0)),
            pl.BlockSpec((H,), lambda i: (0,)),
            pl.BlockSpec((H, HO), lambda i: (0, 0)),
        ],
        out_specs=[
            pl.BlockSpec((RB, HO), lambda i: (i, 0)),
            pl.BlockSpec((RB, HO), lambda i: (i, 0)),
        ],
        out_shape=[
            jax.ShapeDtypeStruct((N, HO), jnp.float32),
            jax.ShapeDtypeStruct((N, HO), jnp.float32),
        ],
    )(p0, p1, m_prev, dis, d2, b_prev, W_next)


def _combine_relu_call(p0, p1, m_prev, dis, d2, b_prev, N):
    H = m_prev.shape[1]
    grid = (N // RB,)

    def body(q0, q1, m_r, dis_r, d2_r, b_r, h_o, gs_o):
        comb = dis_r[...] * (q0[...] + q1[...]) + d2_r[...] * m_r[...] + b_r[...]
        h = jnp.maximum(comb, 0.0)
        h_o[...] = h
        gs_o[...] = dis_r[...] * h

    return pl.pallas_call(
        body,
        grid=grid,
        in_specs=[
            pl.BlockSpec((RB, H), lambda i: (i, 0)),
            pl.BlockSpec((RB, H), lambda i: (i, 0)),
            pl.BlockSpec((RB, H), lambda i: (i, 0)),
            pl.BlockSpec((RB, H), lambda i: (i, 0)),
            pl.BlockSpec((RB, H), lambda i: (i, 0)),
            pl.BlockSpec((H,), lambda i: (0,)),
        ],
        out_specs=[
            pl.BlockSpec((RB, H), lambda i: (i, 0)),
            pl.BlockSpec((RB, H), lambda i: (i, 0)),
        ],
        out_shape=[
            jax.ShapeDtypeStruct((N, H), jnp.float32),
            jax.ShapeDtypeStruct((N, H), jnp.float32),
        ],
    )(p0, p1, m_prev, dis, d2, b_prev)


def _final_call(p0, p1, h2, dis, d2, W4, b4, N):
    H = h2.shape[1]
    HO = W4.shape[1]
    grid = (N // RB,)

    def body(q0, q1, h_r, dis_r, d2_r, w_r, b_r, out_o):
        comb = dis_r[...] * (q0[...] + q1[...]) + d2_r[...] * h_r[...]
        out_o[...] = jnp.dot(comb, w_r[...],
                             preferred_element_type=jnp.float32) + b_r[...]

    return pl.pallas_call(
        body,
        grid=grid,
        in_specs=[
            pl.BlockSpec((RB, H), lambda i: (i, 0)),
            pl.BlockSpec((RB, H), lambda i: (i, 0)),
            pl.BlockSpec((RB, H), lambda i: (i, 0)),
            pl.BlockSpec((RB, H), lambda i: (i, 0)),
            pl.BlockSpec((RB, H), lambda i: (i, 0)),
            pl.BlockSpec((H, HO), lambda i: (0, 0)),
            pl.BlockSpec((HO,), lambda i: (0,)),
        ],
        out_specs=pl.BlockSpec((RB, HO), lambda i: (i, 0)),
        out_shape=jax.ShapeDtypeStruct((N, HO), jnp.float32),
    )(p0, p1, h2, dis, d2, W4, b4)


def kernel(x, edge_index, W1, b1, W2, b2, W3, b3, W4, b4):
    N = x.shape[0]
    H = W1.shape[1]
    E = edge_index.shape[1]

    al = NS * 8
    AR = ((N + 1 + al - 1) // al) * al

    EPT = ((E + NC * NS * CH - 1) // (NC * NS * CH)) * CH
    E_pad = NC * NS * EPT
    TCH = E_pad // CH

    KPP = TCH // NS
    K0 = max(8, min(KPP - 8, (KPP * 7 // 10) // 8 * 8))
    K1 = KPP - K0

    E_stage = (TCH + (K0 - K1)) * CH
    src = edge_index[0]
    dst = edge_index[1]
    if E_stage > E:
        src = jnp.concatenate([src, jnp.zeros((E_stage - E,), jnp.int32)])
        dst = jnp.concatenate([dst, jnp.full((E_stage - E,), N, jnp.int32)])
    src2 = src.reshape(E_stage // CH, CH)
    dst2 = dst.reshape(E_stage // CH, CH)

    deg_k = _make_deg_kernel(E_pad, AR)
    prop_k = _make_prop_kernel(E_pad, AR, H, K0, K1)

    dg0, dg1 = deg_k(dst)
    m1, gs1, dis, d2 = _stats_matmul_call(dg0, dg1, x, W1, N)

    a0, a1 = prop_k(gs1, src2, dst2)
    m2, gs2 = _combine_matmul_call(a0, a1, m1, dis, d2, b1, W2, N, relu=True)

    a0, a1 = prop_k(gs2, src2, dst2)
    m3, gs3 = _combine_matmul_call(a0, a1, m2, dis, d2, b2, W3, N, relu=False)

    a0, a1 = prop_k(gs3, src2, dst2)
    h2, gs4 = _combine_relu_call(a0, a1, m3, dis, d2, b3, N)

    a0, a1 = prop_k(gs4, src2, dst2)
    out = _final_call(a0, a1, h2, dis, d2, W4, b4, N)
    return out


# speedup vs baseline: 1.0367x; 1.0367x over previous
import functools
import math

import jax
import jax.numpy as jnp
from jax import lax
from jax.experimental import pallas as pl
from jax.experimental.pallas import tpu as pltpu
from jax.experimental.pallas import tpu_sc as plsc

NC = 2
NS = 16
CH = 128
RB = 400


def _sc_mesh():
    return plsc.VectorSubcoreMesh(core_axis_name="c", subcore_axis_name="s",
                                  num_cores=NC, num_subcores=NS)


@functools.lru_cache(maxsize=None)
def _make_deg_kernel(E_pad, AR):
    TCH = E_pad // CH
    KPT = TCH // (NC * NS)
    ZR = AR // NS

    @functools.partial(
        pl.kernel,
        out_type=jax.ShapeDtypeStruct((NC, AR, 128), jnp.float32),
        mesh=_sc_mesh(),
        scratch_types=[
            pltpu.VMEM((KPT, CH), jnp.int32),
            pltpu.VMEM((CH, 128), jnp.float32),
            pltpu.VMEM_SHARED((AR, 128), jnp.float32),
        ],
    )
    def deg_kernel(dst_hbm, ones_hbm, zeros_hbm, out_hbm, dst_all, ones_v, acc):
        cid = lax.axis_index("c")
        sid = lax.axis_index("s")
        cbase = (cid * NS + sid) * KPT
        pltpu.sync_copy(dst_hbm.at[pl.ds(cbase, KPT)], dst_all)
        pltpu.sync_copy(ones_hbm, ones_v)

        r0 = sid * ZR
        nfull = ZR // CH
        for t in range(nfull):
            pltpu.sync_copy(zeros_hbm, acc.at[pl.ds(r0 + t * CH, CH)])
        rem = ZR - nfull * CH
        if rem:
            pltpu.sync_copy(zeros_hbm.at[pl.ds(0, rem)],
                            acc.at[pl.ds(r0 + nfull * CH, rem)])
        plsc.subcore_barrier()

        for j in range(KPT):
            pltpu.sync_copy(ones_v, acc.at[dst_all.at[j]], add=True)
        plsc.subcore_barrier()

        pltpu.sync_copy(acc.at[pl.ds(r0, ZR)],
                        out_hbm.at[cid, pl.ds(r0, ZR)])

    return deg_kernel


@functools.lru_cache(maxsize=None)
def _make_prop_kernel(E_pad, AR, D, K0, K1):
    TCH = E_pad // CH
    assert NS * (K0 + K1) == TCH and K0 >= K1
    ZR = AR // NS
    KM = K0

    @functools.partial(
        pl.kernel,
        out_type=jax.ShapeDtypeStruct((NC, AR, D), jnp.float32),
        mesh=_sc_mesh(),
        scratch_types=[
            pltpu.VMEM((KM, CH), jnp.int32),
            pltpu.VMEM((KM, CH), jnp.int32),
            pltpu.VMEM((CH, D), jnp.float32),
            pltpu.VMEM((CH, D), jnp.float32),
            pltpu.VMEM_SHARED((AR, D), jnp.float32),
            pltpu.SemaphoreType.DMA,
        ],
    )
    def prop_kernel(gs_hbm, src_hbm, dst_hbm, zeros_hbm, out_hbm,
                    src_all, dst_all, rows0, rows1, acc, gsem):
        cid = lax.axis_index("c")
        sid = lax.axis_index("s")

        def run(k, cbase):
            pltpu.sync_copy(src_hbm.at[pl.ds(cbase, KM)], src_all)
            pltpu.sync_copy(dst_hbm.at[pl.ds(cbase, KM)], dst_all)

            r0 = sid * ZR
            nfull = ZR // CH
            for t in range(nfull):
                pltpu.sync_copy(zeros_hbm, acc.at[pl.ds(r0 + t * CH, CH)])
            rem = ZR - nfull * CH
            if rem:
                pltpu.sync_copy(zeros_hbm.at[pl.ds(0, rem)],
                                acc.at[pl.ds(r0 + nfull * CH, rem)])
            plsc.subcore_barrier()

            gd = {}
            gd[0] = pltpu.async_copy(gs_hbm.at[src_all.at[0]], rows0, gsem)
            for g in range(k):
                buf = rows0 if g % 2 == 0 else rows1
                nbuf = rows1 if g % 2 == 0 else rows0
                gd[g].wait()
                if g + 1 < k:
                    gd[g + 1] = pltpu.async_copy(
                        gs_hbm.at[src_all.at[g + 1]], nbuf, gsem)
                pltpu.sync_copy(buf, acc.at[dst_all.at[g]], add=True)
            plsc.subcore_barrier()

            r0 = sid * ZR
            pltpu.sync_copy(acc.at[pl.ds(r0, ZR)],
                            out_hbm.at[cid, pl.ds(r0, ZR)])

        if K0 == K1:
            run(K0, (cid * NS + sid) * K0)
        else:
            @pl.when(cid == 0)
            def _():
                run(K0, sid * K0)

            @pl.when(cid == 1)
            def _():
                run(K1, NS * K0 + sid * K1)

    return prop_kernel


def _stats_matmul_call(pp, x, W1, N):
    IN = x.shape[1]
    H = W1.shape[1]
    grid = (N // RB,)

    def body(q0, q1, x_r, w1_r, m1_o, gs1_o, dis_o, d2_o):
        deg = q0[0][:, :1] + q1[0][:, :1] + 1.0
        dis = lax.rsqrt(deg)
        d2 = 1.0 / deg
        dis_b = jnp.broadcast_to(dis, (RB, H))
        m1 = jnp.dot(x_r[...], w1_r[...], preferred_element_type=jnp.float32)
        m1_o[...] = m1
        gs1_o[...] = dis_b * m1
        dis_o[...] = dis_b
        d2_o[...] = jnp.broadcast_to(d2, (RB, H))

    return pl.pallas_call(
        body,
        grid=grid,
        in_specs=[
            pl.BlockSpec((1, RB, 128), lambda i: (0, i, 0)),
            pl.BlockSpec((1, RB, 128), lambda i: (1, i, 0)),
            pl.BlockSpec((RB, IN), lambda i: (i, 0)),
            pl.BlockSpec((IN, H), lambda i: (0, 0)),
        ],
        out_specs=[
            pl.BlockSpec((RB, H), lambda i: (i, 0)),
            pl.BlockSpec((RB, H), lambda i: (i, 0)),
            pl.BlockSpec((RB, H), lambda i: (i, 0)),
            pl.BlockSpec((RB, H), lambda i: (i, 0)),
        ],
        out_shape=[
            jax.ShapeDtypeStruct((N, H), jnp.float32),
            jax.ShapeDtypeStruct((N, H), jnp.float32),
            jax.ShapeDtypeStruct((N, H), jnp.float32),
            jax.ShapeDtypeStruct((N, H), jnp.float32),
        ],
    )(pp, pp, x, W1)


def _combine_matmul_call(pp, m_prev, dis, d2, b_prev, W_next, N, relu):
    H = m_prev.shape[1]
    HO = W_next.shape[1]
    grid = (N // RB,)

    def body(q0, q1, m_r, dis_r, d2_r, b_r, w_r, mn_o, gsn_o):
        comb = dis_r[...] * (q0[0] + q1[0]) + d2_r[...] * m_r[...] + b_r[...]
        if relu:
            comb = jnp.maximum(comb, 0.0)
        mn = jnp.dot(comb, w_r[...], preferred_element_type=jnp.float32)
        mn_o[...] = mn
        gsn_o[...] = dis_r[...] * mn

    return pl.pallas_call(
        body,
        grid=grid,
        in_specs=[
            pl.BlockSpec((1, RB, H), lambda i: (0, i, 0)),
            pl.BlockSpec((1, RB, H), lambda i: (1, i, 0)),
            pl.BlockSpec((RB, H), lambda i: (i, 0)),
            pl.BlockSpec((RB, H), lambda i: (i, 0)),
            pl.BlockSpec((RB, H), lambda i: (i, 0)),
            pl.BlockSpec((H,), lambda i: (0,)),
            pl.BlockSpec((H, HO), lambda i: (0, 0)),
        ],
        out_specs=[
            pl.BlockSpec((RB, HO), lambda i: (i, 0)),
            pl.BlockSpec((RB, HO), lambda i: (i, 0)),
        ],
        out_shape=[
            jax.ShapeDtypeStruct((N, HO), jnp.float32),
            jax.ShapeDtypeStruct((N, HO), jnp.float32),
        ],
    )(pp, pp, m_prev, dis, d2, b_prev, W_next)


def _combine_relu_call(pp, m_prev, dis, d2, b_prev, N):
    H = m_prev.shape[1]
    grid = (N // RB,)

    def body(q0, q1, m_r, dis_r, d2_r, b_r, h_o, gs_o):
        comb = dis_r[...] * (q0[0] + q1[0]) + d2_r[...] * m_r[...] + b_r[...]
        h = jnp.maximum(comb, 0.0)
        h_o[...] = h
        gs_o[...] = dis_r[...] * h

    return pl.pallas_call(
        body,
        grid=grid,
        in_specs=[
            pl.BlockSpec((1, RB, H), lambda i: (0, i, 0)),
            pl.BlockSpec((1, RB, H), lambda i: (1, i, 0)),
            pl.BlockSpec((RB, H), lambda i: (i, 0)),
            pl.BlockSpec((RB, H), lambda i: (i, 0)),
            pl.BlockSpec((RB, H), lambda i: (i, 0)),
            pl.BlockSpec((H,), lambda i: (0,)),
        ],
        out_specs=[
            pl.BlockSpec((RB, H), lambda i: (i, 0)),
            pl.BlockSpec((RB, H), lambda i: (i, 0)),
        ],
        out_shape=[
            jax.ShapeDtypeStruct((N, H), jnp.float32),
            jax.ShapeDtypeStruct((N, H), jnp.float32),
        ],
    )(pp, pp, m_prev, dis, d2, b_prev)


def _final_call(pp, h2, dis, d2, W4, b4, N):
    H = h2.shape[1]
    HO = W4.shape[1]
    grid = (N // RB,)

    def body(q0, q1, h_r, dis_r, d2_r, w_r, b_r, out_o):
        comb = dis_r[...] * (q0[0] + q1[0]) + d2_r[...] * h_r[...]
        out_o[...] = jnp.dot(comb, w_r[...],
                             preferred_element_type=jnp.float32) + b_r[...]

    return pl.pallas_call(
        body,
        grid=grid,
        in_specs=[
            pl.BlockSpec((1, RB, H), lambda i: (0, i, 0)),
            pl.BlockSpec((1, RB, H), lambda i: (1, i, 0)),
            pl.BlockSpec((RB, H), lambda i: (i, 0)),
            pl.BlockSpec((RB, H), lambda i: (i, 0)),
            pl.BlockSpec((RB, H), lambda i: (i, 0)),
            pl.BlockSpec((H, HO), lambda i: (0, 0)),
            pl.BlockSpec((HO,), lambda i: (0,)),
        ],
        out_specs=pl.BlockSpec((RB, HO), lambda i: (i, 0)),
        out_shape=jax.ShapeDtypeStruct((N, HO), jnp.float32),
    )(pp, pp, h2, dis, d2, W4, b4)


def kernel(x, edge_index, W1, b1, W2, b2, W3, b3, W4, b4):
    N = x.shape[0]
    H = W1.shape[1]
    E = edge_index.shape[1]

    al = NS * 8
    AR = ((N + 1 + al - 1) // al) * al

    EPT = ((E + NC * NS * CH - 1) // (NC * NS * CH)) * CH
    E_pad = NC * NS * EPT
    TCH = E_pad // CH

    KPP = TCH // NS
    K0 = KPP // 2 // 8 * 8
    K1 = KPP - K0

    E_stage = (TCH + (K0 - K1)) * CH
    src = edge_index[0]
    dst = edge_index[1]
    if E_stage > E:
        pad = E_stage - E
        src = jnp.concatenate([src, jnp.zeros((pad,), jnp.int32)])
        dst = jnp.concatenate([dst, jnp.full((pad,), N, jnp.int32)])
    src2 = src.reshape(E_stage // CH, CH)
    dst2 = dst.reshape(E_stage // CH, CH)

    ones128 = jnp.ones((CH, 128), jnp.float32)
    zerosH = jnp.zeros((CH, H), jnp.float32)

    deg_k = _make_deg_kernel(E_pad, AR)
    prop_k = _make_prop_kernel(E_pad, AR, H, K0, K1)

    dg = deg_k(dst2, ones128, zerosH)
    m1, gs1, dis, d2 = _stats_matmul_call(dg, x, W1, N)

    acc = prop_k(gs1, src2, dst2, zerosH)
    m2, gs2 = _combine_matmul_call(acc, m1, dis, d2, b1, W2, N, relu=True)

    acc = prop_k(gs2, src2, dst2, zerosH)
    m3, gs3 = _combine_matmul_call(acc, m2, dis, d2, b2, W3, N, relu=False)

    acc = prop_k(gs3, src2, dst2, zerosH)
    h2, gs4 = _combine_relu_call(acc, m3, dis, d2, b3, N)

    acc = prop_k(gs4, src2, dst2, zerosH)
    out = _final_call(acc, h2, dis, d2, W4, b4, N)
    return out


# speedup vs baseline: 2.2786x; 2.1979x over previous
import functools
import math

import jax
import jax.numpy as jnp
from jax import lax
from jax.experimental import pallas as pl
from jax.experimental.pallas import tpu as pltpu
from jax.experimental.pallas import tpu_sc as plsc

NC = 2
NS = 16
CH = 128
RB = 400


def _sc_mesh():
    return plsc.VectorSubcoreMesh(core_axis_name="c", subcore_axis_name="s",
                                  num_cores=NC, num_subcores=NS)


@functools.lru_cache(maxsize=None)
def _make_deg_kernel(E_pad, AR):
    TCH = E_pad // CH
    KPT = TCH // (NC * NS)
    ZR = AR // NS

    @functools.partial(
        pl.kernel,
        out_type=jax.ShapeDtypeStruct((NC, AR, 128), jnp.float32),
        mesh=_sc_mesh(),
        scratch_types=[
            pltpu.VMEM((KPT, CH), jnp.int32),
            pltpu.VMEM((CH, 128), jnp.float32),
            pltpu.VMEM_SHARED((AR, 128), jnp.float32),
        ],
    )
    def deg_kernel(dst_hbm, ones_hbm, zeros_hbm, out_hbm, dst_all, ones_v, acc):
        cid = lax.axis_index("c")
        sid = lax.axis_index("s")
        cbase = (cid * NS + sid) * KPT
        pltpu.sync_copy(dst_hbm.at[pl.ds(cbase, KPT)], dst_all)
        pltpu.sync_copy(ones_hbm, ones_v)

        r0 = sid * ZR
        nfull = ZR // CH
        for t in range(nfull):
            pltpu.sync_copy(zeros_hbm, acc.at[pl.ds(r0 + t * CH, CH)])
        rem = ZR - nfull * CH
        if rem:
            pltpu.sync_copy(zeros_hbm.at[pl.ds(0, rem)],
                            acc.at[pl.ds(r0 + nfull * CH, rem)])
        plsc.subcore_barrier()

        for j in range(KPT):
            pltpu.sync_copy(ones_v, acc.at[dst_all.at[j]], add=True)
        plsc.subcore_barrier()

        pltpu.sync_copy(acc.at[pl.ds(r0, ZR)],
                        out_hbm.at[cid, pl.ds(r0, ZR)])

    return deg_kernel


@functools.lru_cache(maxsize=None)
def _make_prop_kernel(E_pad, AR, D, K0, K1):
    TCH = E_pad // CH
    assert NS * (K0 + K1) == TCH and K0 >= K1
    ZR = AR // NS
    KM = K0

    @functools.partial(
        pl.kernel,
        out_type=jax.ShapeDtypeStruct((NC, AR, D), jnp.float32),
        mesh=_sc_mesh(),
        scratch_types=[
            pltpu.VMEM((KM, CH), jnp.int32),
            pltpu.VMEM((KM, CH), jnp.int32),
            pltpu.VMEM((CH, D), jnp.float32),
            pltpu.VMEM((CH, D), jnp.float32),
            pltpu.VMEM_SHARED((AR, D), jnp.float32),
            pltpu.SemaphoreType.DMA,
        ],
    )
    def prop_kernel(gs_hbm, src_hbm, dst_hbm, zeros_hbm, out_hbm,
                    src_all, dst_all, rows0, rows1, acc, gsem):
        cid = lax.axis_index("c")
        sid = lax.axis_index("s")

        def run(k, cbase):
            pltpu.sync_copy(src_hbm.at[pl.ds(cbase, KM)], src_all)
            pltpu.sync_copy(dst_hbm.at[pl.ds(cbase, KM)], dst_all)

            r0 = sid * ZR
            nfull = ZR // CH
            for t in range(nfull):
                pltpu.sync_copy(zeros_hbm, acc.at[pl.ds(r0 + t * CH, CH)])
            rem = ZR - nfull * CH
            if rem:
                pltpu.sync_copy(zeros_hbm.at[pl.ds(0, rem)],
                                acc.at[pl.ds(r0 + nfull * CH, rem)])
            plsc.subcore_barrier()

            gd = {}
            gd[0] = pltpu.async_copy(gs_hbm.at[src_all.at[0]], rows0, gsem)
            for g in range(k):
                buf = rows0 if g % 2 == 0 else rows1
                nbuf = rows1 if g % 2 == 0 else rows0
                gd[g].wait()
                if g + 1 < k:
                    gd[g + 1] = pltpu.async_copy(
                        gs_hbm.at[src_all.at[g + 1]], nbuf, gsem)
                pltpu.sync_copy(buf, acc.at[dst_all.at[g]], add=True)
            plsc.subcore_barrier()

            r0 = sid * ZR
            pltpu.sync_copy(acc.at[pl.ds(r0, ZR)],
                            out_hbm.at[cid, pl.ds(r0, ZR)])

        if K0 == K1:
            run(K0, (cid * NS + sid) * K0)
        else:
            @pl.when(cid == 0)
            def _():
                run(K0, sid * K0)

            @pl.when(cid == 1)
            def _():
                run(K1, NS * K0 + sid * K1)

    return prop_kernel


def _stats_matmul_call(pp, x, W1, N):
    IN = x.shape[1]
    H = W1.shape[1]
    grid = (N // RB,)

    def body(q0, q1, x_r, w1_r, m1_o, gs1_o, dis_o, d2_o):
        deg = q0[0][:, :1] + q1[0][:, :1] + 1.0
        dis = lax.rsqrt(deg)
        d2 = 1.0 / deg
        dis_b = jnp.broadcast_to(dis, (RB, H))
        m1 = jnp.dot(x_r[...], w1_r[...], preferred_element_type=jnp.float32)
        m1_o[...] = m1
        gs1_o[...] = dis_b * m1
        dis_o[...] = dis_b
        d2_o[...] = jnp.broadcast_to(d2, (RB, H))

    return pl.pallas_call(
        body,
        grid=grid,
        in_specs=[
            pl.BlockSpec((1, RB, 128), lambda i: (0, i, 0)),
            pl.BlockSpec((1, RB, 128), lambda i: (1, i, 0)),
            pl.BlockSpec((RB, IN), lambda i: (i, 0)),
            pl.BlockSpec((IN, H), lambda i: (0, 0)),
        ],
        out_specs=[
            pl.BlockSpec((RB, H), lambda i: (i, 0)),
            pl.BlockSpec((RB, H), lambda i: (i, 0)),
            pl.BlockSpec((RB, H), lambda i: (i, 0)),
            pl.BlockSpec((RB, H), lambda i: (i, 0)),
        ],
        out_shape=[
            jax.ShapeDtypeStruct((N, H), jnp.float32),
            jax.ShapeDtypeStruct((N, H), jnp.float32),
            jax.ShapeDtypeStruct((N, H), jnp.float32),
            jax.ShapeDtypeStruct((N, H), jnp.float32),
        ],
    )(pp, pp, x, W1)


def _combine_matmul_call(pp, m_prev, dis, d2, b_prev, W_next, N, relu):
    H = m_prev.shape[1]
    HO = W_next.shape[1]
    grid = (N // RB,)

    def body(q0, q1, m_r, dis_r, d2_r, b_r, w_r, mn_o, gsn_o):
        comb = dis_r[...] * (q0[0] + q1[0]) + d2_r[...] * m_r[...] + b_r[...]
        if relu:
            comb = jnp.maximum(comb, 0.0)
        mn = jnp.dot(comb, w_r[...], preferred_element_type=jnp.float32)
        mn_o[...] = mn
        gsn_o[...] = dis_r[...] * mn

    return pl.pallas_call(
        body,
        grid=grid,
        in_specs=[
            pl.BlockSpec((1, RB, H), lambda i: (0, i, 0)),
            pl.BlockSpec((1, RB, H), lambda i: (1, i, 0)),
            pl.BlockSpec((RB, H), lambda i: (i, 0)),
            pl.BlockSpec((RB, H), lambda i: (i, 0)),
            pl.BlockSpec((RB, H), lambda i: (i, 0)),
            pl.BlockSpec((H,), lambda i: (0,)),
            pl.BlockSpec((H, HO), lambda i: (0, 0)),
        ],
        out_specs=[
            pl.BlockSpec((RB, HO), lambda i: (i, 0)),
            pl.BlockSpec((RB, HO), lambda i: (i, 0)),
        ],
        out_shape=[
            jax.ShapeDtypeStruct((N, HO), jnp.float32),
            jax.ShapeDtypeStruct((N, HO), jnp.float32),
        ],
    )(pp, pp, m_prev, dis, d2, b_prev, W_next)


def _combine_relu_call(pp, m_prev, dis, d2, b_prev, N):
    H = m_prev.shape[1]
    grid = (N // RB,)

    def body(q0, q1, m_r, dis_r, d2_r, b_r, h_o, gs_o):
        comb = dis_r[...] * (q0[0] + q1[0]) + d2_r[...] * m_r[...] + b_r[...]
        h = jnp.maximum(comb, 0.0)
        h_o[...] = h
        gs_o[...] = dis_r[...] * h

    return pl.pallas_call(
        body,
        grid=grid,
        in_specs=[
            pl.BlockSpec((1, RB, H), lambda i: (0, i, 0)),
            pl.BlockSpec((1, RB, H), lambda i: (1, i, 0)),
            pl.BlockSpec((RB, H), lambda i: (i, 0)),
            pl.BlockSpec((RB, H), lambda i: (i, 0)),
            pl.BlockSpec((RB, H), lambda i: (i, 0)),
            pl.BlockSpec((H,), lambda i: (0,)),
        ],
        out_specs=[
            pl.BlockSpec((RB, H), lambda i: (i, 0)),
            pl.BlockSpec((RB, H), lambda i: (i, 0)),
        ],
        out_shape=[
            jax.ShapeDtypeStruct((N, H), jnp.float32),
            jax.ShapeDtypeStruct((N, H), jnp.float32),
        ],
    )(pp, pp, m_prev, dis, d2, b_prev)


def _final_call(pp, h2, dis, d2, W4, b4, N):
    H = h2.shape[1]
    HO = W4.shape[1]
    grid = (N // RB,)

    def body(q0, q1, h_r, dis_r, d2_r, w_r, b_r, out_o):
        comb = dis_r[...] * (q0[0] + q1[0]) + d2_r[...] * h_r[...]
        out_o[...] = jnp.dot(comb, w_r[...],
                             preferred_element_type=jnp.float32) + b_r[...]

    return pl.pallas_call(
        body,
        grid=grid,
        in_specs=[
            pl.BlockSpec((1, RB, H), lambda i: (0, i, 0)),
            pl.BlockSpec((1, RB, H), lambda i: (1, i, 0)),
            pl.BlockSpec((RB, H), lambda i: (i, 0)),
            pl.BlockSpec((RB, H), lambda i: (i, 0)),
            pl.BlockSpec((RB, H), lambda i: (i, 0)),
            pl.BlockSpec((H, HO), lambda i: (0, 0)),
            pl.BlockSpec((HO,), lambda i: (0,)),
        ],
        out_specs=pl.BlockSpec((RB, HO), lambda i: (i, 0)),
        out_shape=jax.ShapeDtypeStruct((N, HO), jnp.float32),
    )(pp, pp, h2, dis, d2, W4, b4)


def kernel(x, edge_index, W1, b1, W2, b2, W3, b3, W4, b4):
    N = x.shape[0]
    H = W1.shape[1]
    E = edge_index.shape[1]

    al = NS * 8
    AR = ((N + 1 + al - 1) // al) * al

    EPT = ((E + NC * NS * CH - 1) // (NC * NS * CH)) * CH
    E_pad = NC * NS * EPT
    TCH = E_pad // CH

    KPP = TCH // NS
    K0 = KPP // 2 // 8 * 8
    K1 = KPP - K0

    E_stage = (TCH + (K0 - K1)) * CH
    src = edge_index[0]
    dst = edge_index[1]
    if E_stage > E:
        pad = E_stage - E
        ar = jnp.arange(pad, dtype=jnp.int32)
        src = jnp.concatenate([src, (ar * 97) % N])
        dst = jnp.concatenate([dst, N + ar % (AR - N)])
    src2 = src.reshape(E_stage // CH, CH)
    dst2 = dst.reshape(E_stage // CH, CH)

    ones128 = jnp.ones((CH, 128), jnp.float32)
    zerosH = jnp.zeros((CH, H), jnp.float32)

    deg_k = _make_deg_kernel(E_pad, AR)
    prop_k = _make_prop_kernel(E_pad, AR, H, K0, K1)

    dg = deg_k(dst2, ones128, zerosH)
    m1, gs1, dis, d2 = _stats_matmul_call(dg, x, W1, N)

    acc = prop_k(gs1, src2, dst2, zerosH)
    m2, gs2 = _combine_matmul_call(acc, m1, dis, d2, b1, W2, N, relu=True)

    acc = prop_k(gs2, src2, dst2, zerosH)
    m3, gs3 = _combine_matmul_call(acc, m2, dis, d2, b2, W3, N, relu=False)

    acc = prop_k(gs3, src2, dst2, zerosH)
    h2, gs4 = _combine_relu_call(acc, m3, dis, d2, b3, N)

    acc = prop_k(gs4, src2, dst2, zerosH)
    out = _final_call(acc, h2, dis, d2, W4, b4, N)
    return out
